# Initial kernel scaffold; baseline (speedup 1.0000x reference)
import jax
import jax.numpy as jnp
from jax.experimental import pallas as pl


def kernel(features, edge_index, W1, b1, W2, b2):
    raise NotImplementedError("write your pallas kernel here")


# speedup vs baseline: 7.5184x; 7.5184x over previous
import jax
import jax.numpy as jnp
from jax import lax
from jax.experimental import pallas as pl
from jax.experimental.pallas import tpu as pltpu
from jax.experimental.pallas import tpu_sc as plsc

N_NODES = 10000
N_EDGES = 160000
NPAD = 10240
DOFF = 10240
HIST = 2 * NPAD
NC = 2
NS = 16
NW = NC * NS
EDGES_PER_W = N_EDGES // NW
EB = 128
NBLK = N_EDGES // EB
BLK_PER_W = (NBLK + NW - 1) // NW
ROWS_PER_TILE = NPAD // NS

_mesh = lambda: plsc.VectorSubcoreMesh(
    core_axis_name="c", subcore_axis_name="s", num_cores=NC, num_subcores=NS)


def _deg_body(src_hbm, dst_hbm, degp_hbm, sidx_v, didx_v, hist_v):
    c = lax.axis_index("c")
    s = lax.axis_index("s")
    gid = c * NS + s
    base = gid * EDGES_PER_W

    zero16f = jnp.zeros((16,), jnp.float32)
    zero16i = jnp.zeros((16,), jnp.int32)
    ones16 = jnp.ones((16,), jnp.float32)

    sidx_v[pl.ds(4992, 16)] = zero16i
    didx_v[pl.ds(4992, 16)] = zero16i
    pltpu.sync_copy(src_hbm.at[pl.ds(base, EDGES_PER_W)],
                    sidx_v.at[pl.ds(0, EDGES_PER_W)])
    pltpu.sync_copy(dst_hbm.at[pl.ds(base, EDGES_PER_W)],
                    didx_v.at[pl.ds(0, EDGES_PER_W)])

    @pl.loop(0, HIST // 16)
    def _zero(i):
        hist_v[pl.ds(i * 16, 16)] = zero16f

    @pl.loop(0, 312)
    def _scat(k):
        sidx = sidx_v[pl.ds(k * 16, 16)]
        plsc.addupdate_scatter(hist_v, [sidx], ones16)
        didx = didx_v[pl.ds(k * 16, 16)] + DOFF
        plsc.addupdate_scatter(hist_v, [didx], ones16)

    mask8 = lax.iota(jnp.int32, 16) < 8
    sidx = sidx_v[pl.ds(4992, 16)]
    plsc.addupdate_scatter(hist_v, [sidx], ones16, mask=mask8)
    didx = didx_v[pl.ds(4992, 16)] + DOFF
    plsc.addupdate_scatter(hist_v, [didx], ones16, mask=mask8)

    pltpu.sync_copy(hist_v, degp_hbm.at[gid])


def _deg(src, dst):
    return pl.kernel(
        _deg_body,
        out_type=jax.ShapeDtypeStruct((NW, HIST), jnp.float32),
        mesh=_mesh(),
        compiler_params=pltpu.CompilerParams(needs_layout_passes=False),
        scratch_types=[
            pltpu.VMEM((5008,), jnp.int32),
            pltpu.VMEM((5008,), jnp.int32),
            pltpu.VMEM((HIST,), jnp.float32),
        ],
    )(src, dst)


def _make_agg(D):
    nslices = (D + 15) // 16
    offs = [min(k * 16, D - 16) for k in range(nslices)]

    def body(hs_hbm, src_hbm, dst_hbm, out0, out1,
             sidx_v, didx_v, rows_v, zbuf_v, agg_sh, sem):
        c = lax.axis_index("c")
        s = lax.axis_index("s")
        gid = c * NS + s
        zero16f = jnp.zeros((16,), jnp.float32)

        @pl.loop(0, ROWS_PER_TILE)
        def _z(i):
            for off in offs:
                zbuf_v[i, pl.ds(off, 16)] = zero16f

        pltpu.sync_copy(zbuf_v, agg_sh.at[pl.ds(s * ROWS_PER_TILE, ROWS_PER_TILE)])
        plsc.subcore_barrier()

        @pl.loop(0, BLK_PER_W)
        def _blk(j):
            bid = j * NW + gid

            @pl.when(bid < NBLK)
            def _():
                base = bid * EB
                pltpu.sync_copy(src_hbm.at[pl.ds(base, EB)], sidx_v)
                pltpu.sync_copy(dst_hbm.at[pl.ds(base, EB)], didx_v)
                pltpu.async_copy(hs_hbm.at[sidx_v], rows_v, sem).wait()
                pltpu.sync_copy(rows_v, agg_sh.at[didx_v], add=True)

        plsc.subcore_barrier()

        stripe = pl.ds(s * ROWS_PER_TILE, ROWS_PER_TILE)
        pltpu.sync_copy(agg_sh.at[stripe], zbuf_v)

        @pl.when(c == 0)
        def _():
            pltpu.sync_copy(zbuf_v, out0.at[stripe])

        @pl.when(c == 1)
        def _():
            pltpu.sync_copy(zbuf_v, out1.at[stripe])

    def run(hs, src, dst):
        return pl.kernel(
            body,
            out_type=(jax.ShapeDtypeStruct((NPAD, D), jnp.float32),
                      jax.ShapeDtypeStruct((NPAD, D), jnp.float32)),
            mesh=_mesh(),
            compiler_params=pltpu.CompilerParams(use_tc_tiling_on_sc=False),
            scratch_types=[
                pltpu.VMEM((EB,), jnp.int32),
                pltpu.VMEM((EB,), jnp.int32),
                pltpu.VMEM((EB, D), jnp.float32),
                pltpu.VMEM((ROWS_PER_TILE, D), jnp.float32),
                pltpu.VMEM_SHARED((NPAD, D), jnp.float32),
                pltpu.SemaphoreType.DMA,
            ],
        )(hs, src, dst)

    return run


_agg16 = _make_agg(16)
_agg40 = _make_agg(40)


RB = 1280


def _norm_from(degp_blk):
    deg = jnp.sum(degp_blk, axis=0)
    return jnp.where(deg > 0, lax.rsqrt(jnp.maximum(deg, 1.0)), 0.0)


def _t1_body(x_ref, w_ref, degp_ref, o_ref):
    ns = _norm_from(degp_ref[...])
    h = jnp.dot(x_ref[...], w_ref[...], preferred_element_type=jnp.float32)
    o_ref[...] = h * ns[:, None]


def _t1(x, w1, degp):
    return pl.pallas_call(
        _t1_body,
        grid=(NPAD // RB,),
        in_specs=[
            pl.BlockSpec((RB, 256), lambda i: (i, 0)),
            pl.BlockSpec((256, 16), lambda i: (0, 0)),
            pl.BlockSpec((NW, RB), lambda i: (0, i)),
        ],
        out_specs=pl.BlockSpec((RB, 16), lambda i: (i, 0)),
        out_shape=jax.ShapeDtypeStruct((NPAD, 16), jnp.float32),
    )(x, w1, degp)


def _t2_body(p0_ref, p1_ref, dps_ref, dpd_ref, b1_ref, w2_ref, o_ref):
    ns = _norm_from(dps_ref[...])
    nd = _norm_from(dpd_ref[...])
    agg = (p0_ref[...] + p1_ref[...]) * nd[:, None] + b1_ref[...]
    y = jnp.maximum(agg, 0.0)
    h2 = jnp.dot(y, w2_ref[...], preferred_element_type=jnp.float32)
    o_ref[...] = h2 * ns[:, None]


def _t2(p0, p1, degp, b1, w2):
    return pl.pallas_call(
        _t2_body,
        grid=(NPAD // RB,),
        in_specs=[
            pl.BlockSpec((RB, 16), lambda i: (i, 0)),
            pl.BlockSpec((RB, 16), lambda i: (i, 0)),
            pl.BlockSpec((NW, RB), lambda i: (0, i)),
            pl.BlockSpec((NW, RB), lambda i: (0, i + NPAD // RB)),
            pl.BlockSpec((1, 16), lambda i: (0, 0)),
            pl.BlockSpec((16, 40), lambda i: (0, 0)),
        ],
        out_specs=pl.BlockSpec((RB, 40), lambda i: (i, 0)),
        out_shape=jax.ShapeDtypeStruct((NPAD, 40), jnp.float32),
    )(p0, p1, degp, degp, b1, w2)


def _t3_body(q0_ref, q1_ref, dpd_ref, b2_ref, o_ref):
    nd = _norm_from(dpd_ref[...])
    o_ref[...] = (q0_ref[...] + q1_ref[...]) * nd[:, None] + b2_ref[...]


def _t3(q0, q1, degp, b2):
    return pl.pallas_call(
        _t3_body,
        grid=(NPAD // RB,),
        in_specs=[
            pl.BlockSpec((RB, 40), lambda i: (i, 0)),
            pl.BlockSpec((RB, 40), lambda i: (i, 0)),
            pl.BlockSpec((NW, RB), lambda i: (0, i + NPAD // RB)),
            pl.BlockSpec((1, 40), lambda i: (0, 0)),
        ],
        out_specs=pl.BlockSpec((RB, 40), lambda i: (i, 0)),
        out_shape=jax.ShapeDtypeStruct((NPAD, 40), jnp.float32),
    )(q0, q1, degp, b2)


def kernel(features, edge_index, W1, b1, W2, b2):
    src = edge_index[0].astype(jnp.int32)
    dst = edge_index[1].astype(jnp.int32)

    degp = _deg(src, dst)
    hs1 = _t1(features, W1, degp)
    p0, p1 = _agg16(hs1, src, dst)
    hs2 = _t2(p0, p1, degp, jnp.reshape(b1, (1, 16)), W2)
    q0, q1 = _agg40(hs2, src, dst)
    out = _t3(q0, q1, degp, jnp.reshape(b2, (1, 40)))
    return out[:N_NODES]


# speedup vs baseline: 13.7646x; 1.8308x over previous
import jax
import jax.numpy as jnp
from jax import lax
from jax.experimental import pallas as pl
from jax.experimental.pallas import tpu as pltpu
from jax.experimental.pallas import tpu_sc as plsc

N_NODES = 10000
N_EDGES = 160000
NPAD = 10240
DOFF = 10240
HIST = 2 * NPAD
NC = 2
NS = 16
NW = NC * NS
EDGES_PER_W = N_EDGES // NW
EB = 125
BLK_PER_W = EDGES_PER_W // EB
ROWS_PER_TILE = NPAD // NS
NBUF = 4

_mesh = lambda: plsc.VectorSubcoreMesh(
    core_axis_name="c", subcore_axis_name="s", num_cores=NC, num_subcores=NS)


def _deg_body(src_hbm, dst_hbm, degp_hbm, sidx_v, didx_v, hist_v):
    c = lax.axis_index("c")
    s = lax.axis_index("s")
    gid = c * NS + s
    base = gid * EDGES_PER_W

    zero16f = jnp.zeros((16,), jnp.float32)
    zero16i = jnp.zeros((16,), jnp.int32)
    ones16 = jnp.ones((16,), jnp.float32)

    sidx_v[pl.ds(4992, 16)] = zero16i
    didx_v[pl.ds(4992, 16)] = zero16i
    pltpu.sync_copy(src_hbm.at[pl.ds(base, EDGES_PER_W)],
                    sidx_v.at[pl.ds(0, EDGES_PER_W)])
    pltpu.sync_copy(dst_hbm.at[pl.ds(base, EDGES_PER_W)],
                    didx_v.at[pl.ds(0, EDGES_PER_W)])

    @pl.loop(0, HIST // 16)
    def _zero(i):
        hist_v[pl.ds(i * 16, 16)] = zero16f

    @pl.loop(0, 312)
    def _scat(k):
        sidx = sidx_v[pl.ds(k * 16, 16)]
        plsc.addupdate_scatter(hist_v, [sidx], ones16)
        didx = didx_v[pl.ds(k * 16, 16)] + DOFF
        plsc.addupdate_scatter(hist_v, [didx], ones16)

    mask8 = lax.iota(jnp.int32, 16) < 8
    sidx = sidx_v[pl.ds(4992, 16)]
    plsc.addupdate_scatter(hist_v, [sidx], ones16, mask=mask8)
    didx = didx_v[pl.ds(4992, 16)] + DOFF
    plsc.addupdate_scatter(hist_v, [didx], ones16, mask=mask8)

    pltpu.sync_copy(hist_v, degp_hbm.at[gid])


def _deg(src, dst):
    return pl.kernel(
        _deg_body,
        out_type=jax.ShapeDtypeStruct((NW, HIST), jnp.float32),
        mesh=_mesh(),
        compiler_params=pltpu.CompilerParams(needs_layout_passes=False),
        scratch_types=[
            pltpu.VMEM((5008,), jnp.int32),
            pltpu.VMEM((5008,), jnp.int32),
            pltpu.VMEM((HIST,), jnp.float32),
        ],
    )(src, dst)


def _make_agg(D):
    nslices = (D + 15) // 16
    offs = [min(k * 16, D - 16) for k in range(nslices)]

    def body(hs_hbm, src_hbm, dst_hbm, out0, out1,
             sidx_v, didx_v, rows_v, zbuf_v, agg_sh, gsem, ssem):
        c = lax.axis_index("c")
        s = lax.axis_index("s")
        gid = c * NS + s
        zero16f = jnp.zeros((16,), jnp.float32)

        row0 = gid * BLK_PER_W
        pltpu.sync_copy(src_hbm.at[pl.ds(row0, BLK_PER_W)], sidx_v)
        pltpu.sync_copy(dst_hbm.at[pl.ds(row0, BLK_PER_W)], didx_v)

        @pl.loop(0, ROWS_PER_TILE)
        def _z(i):
            for off in offs:
                zbuf_v[i, pl.ds(off, 16)] = zero16f

        pltpu.sync_copy(zbuf_v, agg_sh.at[pl.ds(s * ROWS_PER_TILE, ROWS_PER_TILE)])
        plsc.subcore_barrier()

        def gather_start(j, b):
            pltpu.async_copy(hs_hbm.at[sidx_v.at[j]], rows_v.at[b], gsem.at[b])

        def gather_wait(j, b):
            pltpu.make_async_copy(
                hs_hbm.at[sidx_v.at[j]], rows_v.at[b], gsem.at[b]).wait()

        def scatter_start(j, b):
            pltpu.async_copy(
                rows_v.at[b], agg_sh.at[didx_v.at[j]], ssem.at[b], add=True)

        def scatter_wait(j, b):
            pltpu.make_async_copy(
                rows_v.at[b], agg_sh.at[didx_v.at[j]], ssem.at[b]).wait()

        for b in range(NBUF - 1):
            gather_start(b, b)

        @pl.loop(0, BLK_PER_W, step=NBUF)
        def _blk(j):
            for b in range(NBUF):
                jj = j + b
                gather_wait(jj, b)
                scatter_start(jj, b)
                nb = (b + NBUF - 1) % NBUF

                @pl.when(jj >= 1)
                def _():
                    scatter_wait(jj - 1, nb)

                @pl.when(jj + NBUF - 1 < BLK_PER_W)
                def _():
                    gather_start(jj + NBUF - 1, nb)

        scatter_wait(BLK_PER_W - 1, (BLK_PER_W - 1) % NBUF)
        plsc.subcore_barrier()

        stripe = pl.ds(s * ROWS_PER_TILE, ROWS_PER_TILE)
        pltpu.sync_copy(agg_sh.at[stripe], zbuf_v)

        @pl.when(c == 0)
        def _():
            pltpu.sync_copy(zbuf_v, out0.at[stripe])

        @pl.when(c == 1)
        def _():
            pltpu.sync_copy(zbuf_v, out1.at[stripe])

    def run(hs, src, dst):
        return pl.kernel(
            body,
            out_type=(jax.ShapeDtypeStruct((NPAD, D), jnp.float32),
                      jax.ShapeDtypeStruct((NPAD, D), jnp.float32)),
            mesh=_mesh(),
            compiler_params=pltpu.CompilerParams(use_tc_tiling_on_sc=False),
            scratch_types=[
                pltpu.VMEM((BLK_PER_W, EB), jnp.int32),
                pltpu.VMEM((BLK_PER_W, EB), jnp.int32),
                pltpu.VMEM((NBUF, EB, D), jnp.float32),
                pltpu.VMEM((ROWS_PER_TILE, D), jnp.float32),
                pltpu.VMEM_SHARED((NPAD, D), jnp.float32),
                pltpu.SemaphoreType.DMA((NBUF,)),
                pltpu.SemaphoreType.DMA((NBUF,)),
            ],
        )(hs, src, dst)

    return run


_agg16 = _make_agg(16)
_agg40 = _make_agg(40)


RB = 1280


def _norm_from(degp_blk):
    deg = jnp.sum(degp_blk, axis=0)
    return jnp.where(deg > 0, lax.rsqrt(jnp.maximum(deg, 1.0)), 0.0)


def _t1_body(x_ref, w_ref, degp_ref, o_ref):
    ns = _norm_from(degp_ref[...])
    h = jnp.dot(x_ref[...], w_ref[...], preferred_element_type=jnp.float32)
    o_ref[...] = h * ns[:, None]


def _t1(x, w1, degp):
    return pl.pallas_call(
        _t1_body,
        grid=(NPAD // RB,),
        in_specs=[
            pl.BlockSpec((RB, 256), lambda i: (i, 0)),
            pl.BlockSpec((256, 16), lambda i: (0, 0)),
            pl.BlockSpec((NW, RB), lambda i: (0, i)),
        ],
        out_specs=pl.BlockSpec((RB, 16), lambda i: (i, 0)),
        out_shape=jax.ShapeDtypeStruct((NPAD, 16), jnp.float32),
    )(x, w1, degp)


def _t2_body(p0_ref, p1_ref, dps_ref, dpd_ref, b1_ref, w2_ref, o_ref):
    ns = _norm_from(dps_ref[...])
    nd = _norm_from(dpd_ref[...])
    agg = (p0_ref[...] + p1_ref[...]) * nd[:, None] + b1_ref[...]
    y = jnp.maximum(agg, 0.0)
    h2 = jnp.dot(y, w2_ref[...], preferred_element_type=jnp.float32)
    o_ref[...] = h2 * ns[:, None]


def _t2(p0, p1, degp, b1, w2):
    return pl.pallas_call(
        _t2_body,
        grid=(NPAD // RB,),
        in_specs=[
            pl.BlockSpec((RB, 16), lambda i: (i, 0)),
            pl.BlockSpec((RB, 16), lambda i: (i, 0)),
            pl.BlockSpec((NW, RB), lambda i: (0, i)),
            pl.BlockSpec((NW, RB), lambda i: (0, i + NPAD // RB)),
            pl.BlockSpec((1, 16), lambda i: (0, 0)),
            pl.BlockSpec((16, 40), lambda i: (0, 0)),
        ],
        out_specs=pl.BlockSpec((RB, 40), lambda i: (i, 0)),
        out_shape=jax.ShapeDtypeStruct((NPAD, 40), jnp.float32),
    )(p0, p1, degp, degp, b1, w2)


def _t3_body(q0_ref, q1_ref, dpd_ref, b2_ref, o_ref):
    nd = _norm_from(dpd_ref[...])
    o_ref[...] = (q0_ref[...] + q1_ref[...]) * nd[:, None] + b2_ref[...]


def _t3(q0, q1, degp, b2):
    return pl.pallas_call(
        _t3_body,
        grid=(NPAD // RB,),
        in_specs=[
            pl.BlockSpec((RB, 40), lambda i: (i, 0)),
            pl.BlockSpec((RB, 40), lambda i: (i, 0)),
            pl.BlockSpec((NW, RB), lambda i: (0, i + NPAD // RB)),
            pl.BlockSpec((1, 40), lambda i: (0, 0)),
        ],
        out_specs=pl.BlockSpec((RB, 40), lambda i: (i, 0)),
        out_shape=jax.ShapeDtypeStruct((NPAD, 40), jnp.float32),
    )(q0, q1, degp, b2)


def kernel(features, edge_index, W1, b1, W2, b2):
    src = edge_index[0].astype(jnp.int32)
    dst = edge_index[1].astype(jnp.int32)
    src2d = jnp.reshape(src, (NW * BLK_PER_W, EB))
    dst2d = jnp.reshape(dst, (NW * BLK_PER_W, EB))

    degp = _deg(src, dst)
    hs1 = _t1(features, W1, degp)
    p0, p1 = _agg16(hs1, src2d, dst2d)
    hs2 = _t2(p0, p1, degp, jnp.reshape(b1, (1, 16)), W2)
    q0, q1 = _agg40(hs2, src2d, dst2d)
    out = _t3(q0, q1, degp, jnp.reshape(b2, (1, 40)))
    return out[:N_NODES]


# speedup vs baseline: 15.0047x; 1.0901x over previous
import jax
import jax.numpy as jnp
from jax import lax
from jax.experimental import pallas as pl
from jax.experimental.pallas import tpu as pltpu
from jax.experimental.pallas import tpu_sc as plsc

N_NODES = 10000
N_EDGES = 160000
NPAD = 10240
DOFF = 10240
HIST = 2 * NPAD
NC = 2
NS = 16
NW = NC * NS
EB = 128
EROWS = 1280
BLK_PER_W = EROWS // NW
ROWS_PER_TILE = NPAD // NS
NBUF = 4

_mesh = lambda: plsc.VectorSubcoreMesh(
    core_axis_name="c", subcore_axis_name="s", num_cores=NC, num_subcores=NS)


def _deg_body(e_hbm, degp_hbm, sidx_v, didx_v, hist_v, sems):
    c = lax.axis_index("c")
    s = lax.axis_index("s")
    gid = c * NS + s
    row0 = gid * BLK_PER_W

    zero16f = jnp.zeros((16,), jnp.float32)
    ones16 = jnp.ones((16,), jnp.float32)

    pltpu.async_copy(e_hbm.at[0, pl.ds(row0, BLK_PER_W)], sidx_v, sems.at[0])
    pltpu.async_copy(e_hbm.at[1, pl.ds(row0, BLK_PER_W)], didx_v, sems.at[1])

    @pl.loop(0, HIST // 16)
    def _zero(i):
        hist_v[pl.ds(i * 16, 16)] = zero16f

    pltpu.make_async_copy(
        e_hbm.at[0, pl.ds(row0, BLK_PER_W)], sidx_v, sems.at[0]).wait()
    pltpu.make_async_copy(
        e_hbm.at[1, pl.ds(row0, BLK_PER_W)], didx_v, sems.at[1]).wait()

    @pl.loop(0, BLK_PER_W)
    def _scat(r):
        for k in range(EB // 16):
            sidx = sidx_v[r, pl.ds(k * 16, 16)]
            plsc.addupdate_scatter(hist_v, [sidx], ones16)
            didx = didx_v[r, pl.ds(k * 16, 16)] + DOFF
            plsc.addupdate_scatter(hist_v, [didx], ones16)

    pltpu.sync_copy(hist_v, degp_hbm.at[gid])


def _deg(e3):
    return pl.kernel(
        _deg_body,
        out_type=jax.ShapeDtypeStruct((NW, HIST), jnp.float32),
        mesh=_mesh(),
        compiler_params=pltpu.CompilerParams(needs_layout_passes=False),
        scratch_types=[
            pltpu.VMEM((BLK_PER_W, EB), jnp.int32),
            pltpu.VMEM((BLK_PER_W, EB), jnp.int32),
            pltpu.VMEM((HIST,), jnp.float32),
            pltpu.SemaphoreType.DMA((2,)),
        ],
    )(e3)


def _make_agg(D):
    nslices = (D + 15) // 16
    offs = [min(k * 16, D - 16) for k in range(nslices)]

    def body(hs_hbm, e_hbm, out0, out1,
             sidx_v, didx_v, rows_v, zbuf_v, agg_sh, isem, gsem, ssem):
        c = lax.axis_index("c")
        s = lax.axis_index("s")
        gid = c * NS + s
        row0 = gid * BLK_PER_W
        zero16f = jnp.zeros((16,), jnp.float32)

        pltpu.async_copy(e_hbm.at[0, pl.ds(row0, BLK_PER_W)], sidx_v, isem.at[0])
        pltpu.async_copy(e_hbm.at[1, pl.ds(row0, BLK_PER_W)], didx_v, isem.at[1])

        @pl.loop(0, ROWS_PER_TILE)
        def _z(i):
            for off in offs:
                zbuf_v[i, pl.ds(off, 16)] = zero16f

        pltpu.sync_copy(zbuf_v, agg_sh.at[pl.ds(s * ROWS_PER_TILE, ROWS_PER_TILE)])
        pltpu.make_async_copy(
            e_hbm.at[0, pl.ds(row0, BLK_PER_W)], sidx_v, isem.at[0]).wait()
        pltpu.make_async_copy(
            e_hbm.at[1, pl.ds(row0, BLK_PER_W)], didx_v, isem.at[1]).wait()
        plsc.subcore_barrier()

        def gather_start(j, b):
            pltpu.async_copy(hs_hbm.at[sidx_v.at[j]], rows_v.at[b], gsem.at[b])

        def gather_wait(j, b):
            pltpu.make_async_copy(
                hs_hbm.at[sidx_v.at[j]], rows_v.at[b], gsem.at[b]).wait()

        def scatter_start(j, b):
            pltpu.async_copy(
                rows_v.at[b], agg_sh.at[didx_v.at[j]], ssem.at[b], add=True)

        def scatter_wait(j, b):
            pltpu.make_async_copy(
                rows_v.at[b], agg_sh.at[didx_v.at[j]], ssem.at[b]).wait()

        for b in range(NBUF - 1):
            gather_start(b, b)

        @pl.loop(0, BLK_PER_W, step=NBUF)
        def _blk(j):
            for b in range(NBUF):
                jj = j + b
                gather_wait(jj, b)
                scatter_start(jj, b)
                nb = (b + NBUF - 1) % NBUF

                @pl.when(jj >= 1)
                def _():
                    scatter_wait(jj - 1, nb)

                @pl.when(jj + NBUF - 1 < BLK_PER_W)
                def _():
                    gather_start(jj + NBUF - 1, nb)

        scatter_wait(BLK_PER_W - 1, (BLK_PER_W - 1) % NBUF)
        plsc.subcore_barrier()

        stripe = pl.ds(s * ROWS_PER_TILE, ROWS_PER_TILE)
        pltpu.sync_copy(agg_sh.at[stripe], zbuf_v)

        @pl.when(c == 0)
        def _():
            pltpu.sync_copy(zbuf_v, out0.at[stripe])

        @pl.when(c == 1)
        def _():
            pltpu.sync_copy(zbuf_v, out1.at[stripe])

    def run(hs, e3):
        return pl.kernel(
            body,
            out_type=(jax.ShapeDtypeStruct((NPAD, D), jnp.float32),
                      jax.ShapeDtypeStruct((NPAD, D), jnp.float32)),
            mesh=_mesh(),
            compiler_params=pltpu.CompilerParams(use_tc_tiling_on_sc=False),
            scratch_types=[
                pltpu.VMEM((BLK_PER_W, EB), jnp.int32),
                pltpu.VMEM((BLK_PER_W, EB), jnp.int32),
                pltpu.VMEM((NBUF, EB, D), jnp.float32),
                pltpu.VMEM((ROWS_PER_TILE, D), jnp.float32),
                pltpu.VMEM_SHARED((NPAD, D), jnp.float32),
                pltpu.SemaphoreType.DMA((2,)),
                pltpu.SemaphoreType.DMA((NBUF,)),
                pltpu.SemaphoreType.DMA((NBUF,)),
            ],
        )(hs, e3)

    return run


_agg16 = _make_agg(16)
_agg40 = _make_agg(40)


RB = 2560


def _norm_from(degp_blk):
    deg = jnp.sum(degp_blk, axis=0)
    return jnp.where(deg > 0, lax.rsqrt(jnp.maximum(deg, 1.0)), 0.0)


def _rowmask(i, rows):
    row = i * RB + lax.broadcasted_iota(jnp.int32, (rows, 1), 0)
    return row < N_NODES


def _t1_body(x_ref, w_ref, degp_ref, o_ref):
    i = pl.program_id(0)
    ns = _norm_from(degp_ref[...])
    h = jnp.dot(x_ref[...], w_ref[...], preferred_element_type=jnp.float32)
    o_ref[...] = jnp.where(_rowmask(i, RB), h * ns[:, None], 0.0)


def _t1(x, w1, degp):
    return pl.pallas_call(
        _t1_body,
        grid=(NPAD // RB,),
        in_specs=[
            pl.BlockSpec((RB, 256), lambda i: (i, 0)),
            pl.BlockSpec((256, 16), lambda i: (0, 0)),
            pl.BlockSpec((NW, RB), lambda i: (0, i)),
        ],
        out_specs=pl.BlockSpec((RB, 16), lambda i: (i, 0)),
        out_shape=jax.ShapeDtypeStruct((NPAD, 16), jnp.float32),
    )(x, w1, degp)


def _t2_body(p0_ref, p1_ref, dps_ref, dpd_ref, b1_ref, w2_ref, o_ref):
    i = pl.program_id(0)
    ns = _norm_from(dps_ref[...])
    nd = _norm_from(dpd_ref[...])
    agg = (p0_ref[...] + p1_ref[...]) * nd[:, None] + b1_ref[...]
    y = jnp.maximum(agg, 0.0)
    h2 = jnp.dot(y, w2_ref[...], preferred_element_type=jnp.float32)
    o_ref[...] = jnp.where(_rowmask(i, RB), h2 * ns[:, None], 0.0)


def _t2(p0, p1, degp, b1, w2):
    return pl.pallas_call(
        _t2_body,
        grid=(NPAD // RB,),
        in_specs=[
            pl.BlockSpec((RB, 16), lambda i: (i, 0)),
            pl.BlockSpec((RB, 16), lambda i: (i, 0)),
            pl.BlockSpec((NW, RB), lambda i: (0, i)),
            pl.BlockSpec((NW, RB), lambda i: (0, i + NPAD // RB)),
            pl.BlockSpec((1, 16), lambda i: (0, 0)),
            pl.BlockSpec((16, 40), lambda i: (0, 0)),
        ],
        out_specs=pl.BlockSpec((RB, 40), lambda i: (i, 0)),
        out_shape=jax.ShapeDtypeStruct((NPAD, 40), jnp.float32),
    )(p0, p1, degp, degp, b1, w2)


def _t3_body(q0_ref, q1_ref, dpd_ref, b2_ref, o_ref):
    nd = _norm_from(dpd_ref[...])
    o_ref[...] = (q0_ref[...] + q1_ref[...]) * nd[:, None] + b2_ref[...]


def _t3(q0, q1, degp, b2):
    return pl.pallas_call(
        _t3_body,
        grid=(NPAD // RB,),
        in_specs=[
            pl.BlockSpec((RB, 40), lambda i: (i, 0)),
            pl.BlockSpec((RB, 40), lambda i: (i, 0)),
            pl.BlockSpec((NW, RB), lambda i: (0, i + NPAD // RB)),
            pl.BlockSpec((1, 40), lambda i: (0, 0)),
        ],
        out_specs=pl.BlockSpec((RB, 40), lambda i: (i, 0)),
        out_shape=jax.ShapeDtypeStruct((N_NODES, 40), jnp.float32),
    )(q0, q1, degp, b2)


def kernel(features, edge_index, W1, b1, W2, b2):
    e2d = jnp.reshape(edge_index.astype(jnp.int32), (2, EROWS, 125))
    pad3 = (N_NODES + (jnp.arange(EROWS * 3, dtype=jnp.int32) % (NPAD - N_NODES))
            ).reshape(EROWS, 3)
    e3 = jnp.concatenate(
        [e2d, jnp.broadcast_to(pad3[None], (2, EROWS, 3))], axis=2)

    degp = _deg(e3)
    hs1 = _t1(features, W1, degp)
    p0, p1 = _agg16(hs1, e3)
    hs2 = _t2(p0, p1, degp, jnp.reshape(b1, (1, 16)), W2)
    q0, q1 = _agg40(hs2, e3)
    return _t3(q0, q1, degp, jnp.reshape(b2, (1, 40)))


# speedup vs baseline: 15.5920x; 1.0391x over previous
import jax
import jax.numpy as jnp
from jax import lax
from jax.experimental import pallas as pl
from jax.experimental.pallas import tpu as pltpu
from jax.experimental.pallas import tpu_sc as plsc

N_NODES = 10000
N_EDGES = 160000
NPAD = 10240
DOFF = 10240
HIST = 2 * NPAD
NC = 2
NS = 16
NW = NC * NS
EB = 128
EROWS = 1280
BLK_PER_W = EROWS // NW
ROWS_PER_TILE = NPAD // NS
NBUF = 4

_mesh = lambda: plsc.VectorSubcoreMesh(
    core_axis_name="c", subcore_axis_name="s", num_cores=NC, num_subcores=NS)


def _deg_body(e_hbm, degp_hbm, sidx_v, didx_v, hist_v, sems):
    c = lax.axis_index("c")
    s = lax.axis_index("s")
    gid = c * NS + s
    row0 = gid * BLK_PER_W

    zero16f = jnp.zeros((16,), jnp.float32)
    ones16 = jnp.ones((16,), jnp.float32)

    pltpu.async_copy(e_hbm.at[0, pl.ds(row0, BLK_PER_W)], sidx_v, sems.at[0])
    pltpu.async_copy(e_hbm.at[1, pl.ds(row0, BLK_PER_W)], didx_v, sems.at[1])

    @pl.loop(0, HIST // 16)
    def _zero(i):
        hist_v[pl.ds(i * 16, 16)] = zero16f

    pltpu.make_async_copy(
        e_hbm.at[0, pl.ds(row0, BLK_PER_W)], sidx_v, sems.at[0]).wait()
    pltpu.make_async_copy(
        e_hbm.at[1, pl.ds(row0, BLK_PER_W)], didx_v, sems.at[1]).wait()

    @pl.loop(0, BLK_PER_W)
    def _scat(r):
        for k in range(EB // 16):
            sidx = sidx_v[r, pl.ds(k * 16, 16)]
            plsc.addupdate_scatter(hist_v, [sidx], ones16)
            didx = didx_v[r, pl.ds(k * 16, 16)] + DOFF
            plsc.addupdate_scatter(hist_v, [didx], ones16)

    pltpu.sync_copy(hist_v, degp_hbm.at[gid])


def _deg(e3):
    return pl.kernel(
        _deg_body,
        out_type=jax.ShapeDtypeStruct((NW, HIST), jnp.float32),
        mesh=_mesh(),
        compiler_params=pltpu.CompilerParams(needs_layout_passes=False),
        scratch_types=[
            pltpu.VMEM((BLK_PER_W, EB), jnp.int32),
            pltpu.VMEM((BLK_PER_W, EB), jnp.int32),
            pltpu.VMEM((HIST,), jnp.float32),
            pltpu.SemaphoreType.DMA((2,)),
        ],
    )(e3)


def _make_agg(D):
    nslices = (D + 15) // 16
    offs = [min(k * 16, D - 16) for k in range(nslices)]

    def body(hs_hbm, e_hbm, out0, out1,
             sidx_v, didx_v, rows_v, zbuf_v, agg_sh, isem, gsem, ssem):
        c = lax.axis_index("c")
        s = lax.axis_index("s")
        gid = c * NS + s
        row0 = gid * BLK_PER_W
        zero16f = jnp.zeros((16,), jnp.float32)

        pltpu.async_copy(e_hbm.at[0, pl.ds(row0, BLK_PER_W)], sidx_v, isem.at[0])
        pltpu.async_copy(e_hbm.at[1, pl.ds(row0, BLK_PER_W)], didx_v, isem.at[1])

        @pl.loop(0, ROWS_PER_TILE)
        def _z(i):
            for off in offs:
                zbuf_v[i, pl.ds(off, 16)] = zero16f

        pltpu.sync_copy(zbuf_v, agg_sh.at[pl.ds(s * ROWS_PER_TILE, ROWS_PER_TILE)])
        pltpu.make_async_copy(
            e_hbm.at[0, pl.ds(row0, BLK_PER_W)], sidx_v, isem.at[0]).wait()
        pltpu.make_async_copy(
            e_hbm.at[1, pl.ds(row0, BLK_PER_W)], didx_v, isem.at[1]).wait()
        plsc.subcore_barrier()

        def gather_start(j, b):
            pltpu.async_copy(hs_hbm.at[sidx_v.at[j]], rows_v.at[b], gsem.at[b])

        def gather_wait(j, b):
            pltpu.make_async_copy(
                hs_hbm.at[sidx_v.at[j]], rows_v.at[b], gsem.at[b]).wait()

        def scatter_start(j, b):
            pltpu.async_copy(
                rows_v.at[b], agg_sh.at[didx_v.at[j]], ssem.at[b], add=True)

        def scatter_wait(j, b):
            pltpu.make_async_copy(
                rows_v.at[b], agg_sh.at[didx_v.at[j]], ssem.at[b]).wait()

        for b in range(NBUF - 1):
            gather_start(b, b)

        @pl.loop(0, BLK_PER_W, step=NBUF)
        def _blk(j):
            for b in range(NBUF):
                jj = j + b
                gather_wait(jj, b)
                scatter_start(jj, b)
                nb = (b + NBUF - 1) % NBUF

                @pl.when(jj >= 1)
                def _():
                    scatter_wait(jj - 1, nb)

                @pl.when(jj + NBUF - 1 < BLK_PER_W)
                def _():
                    gather_start(jj + NBUF - 1, nb)

        scatter_wait(BLK_PER_W - 1, (BLK_PER_W - 1) % NBUF)
        plsc.subcore_barrier()

        stripe = pl.ds(s * ROWS_PER_TILE, ROWS_PER_TILE)
        pltpu.sync_copy(agg_sh.at[stripe], zbuf_v)

        @pl.when(c == 0)
        def _():
            pltpu.sync_copy(zbuf_v, out0.at[stripe])

        @pl.when(c == 1)
        def _():
            pltpu.sync_copy(zbuf_v, out1.at[stripe])

    def run(hs, e3):
        return pl.kernel(
            body,
            out_type=(jax.ShapeDtypeStruct((NPAD, D), jnp.float32),
                      jax.ShapeDtypeStruct((NPAD, D), jnp.float32)),
            mesh=_mesh(),
            compiler_params=pltpu.CompilerParams(use_tc_tiling_on_sc=False),
            scratch_types=[
                pltpu.VMEM((BLK_PER_W, EB), jnp.int32),
                pltpu.VMEM((BLK_PER_W, EB), jnp.int32),
                pltpu.VMEM((NBUF, EB, D), jnp.float32),
                pltpu.VMEM((ROWS_PER_TILE, D), jnp.float32),
                pltpu.VMEM_SHARED((NPAD, D), jnp.float32),
                pltpu.SemaphoreType.DMA((2,)),
                pltpu.SemaphoreType.DMA((NBUF,)),
                pltpu.SemaphoreType.DMA((NBUF,)),
            ],
        )(hs, e3)

    return run


_agg16 = _make_agg(16)
_agg40 = _make_agg(40)


RB = 2560


def _norm_from(degp_blk):
    deg = jnp.sum(degp_blk, axis=0)
    return jnp.where(deg > 0, lax.rsqrt(jnp.maximum(deg, 1.0)), 0.0)


def _rowmask(i, rows):
    row = i * RB + lax.broadcasted_iota(jnp.int32, (rows, 1), 0)
    return row < N_NODES


R16 = RB * 16 // 128
R40 = RB * 40 // 128


def _t1_body(x_ref, w_ref, degp_ref, o_ref):
    i = pl.program_id(0)
    ns = _norm_from(degp_ref[...])
    h = jnp.dot(x_ref[...], w_ref[...], preferred_element_type=jnp.float32)
    hs = jnp.where(_rowmask(i, RB), h * ns[:, None], 0.0)
    hs3 = jnp.reshape(hs, (R16, 8, 16))
    for g in range(8):
        o_ref[:, g * 16:(g + 1) * 16] = hs3[:, g, :]


def _t1(x, w1, degp):
    return pl.pallas_call(
        _t1_body,
        grid=(NPAD // RB,),
        in_specs=[
            pl.BlockSpec((RB, 256), lambda i: (i, 0)),
            pl.BlockSpec((256, 16), lambda i: (0, 0)),
            pl.BlockSpec((NW, RB), lambda i: (0, i)),
        ],
        out_specs=pl.BlockSpec((R16, 128), lambda i: (i, 0)),
        out_shape=jax.ShapeDtypeStruct((NPAD * 16 // 128, 128), jnp.float32),
    )(x, w1, degp)


def _t2_body(p0_ref, p1_ref, dps_ref, dpd_ref, b1_ref, w2_ref, o_ref):
    i = pl.program_id(0)
    ns = _norm_from(dps_ref[...])
    nd = _norm_from(dpd_ref[...])
    psum = p0_ref[...] + p1_ref[...]
    p = jnp.reshape(
        jnp.stack([psum[:, g * 16:(g + 1) * 16] for g in range(8)], axis=1),
        (RB, 16))
    agg = p * nd[:, None] + b1_ref[...]
    y = jnp.maximum(agg, 0.0)
    h2 = jnp.dot(y, w2_ref[...], preferred_element_type=jnp.float32)
    o_ref[...] = jnp.where(_rowmask(i, RB), h2 * ns[:, None], 0.0)


def _t2(p0, p1, degp, b1, w2):
    return pl.pallas_call(
        _t2_body,
        grid=(NPAD // RB,),
        in_specs=[
            pl.BlockSpec((R16, 128), lambda i: (i, 0)),
            pl.BlockSpec((R16, 128), lambda i: (i, 0)),
            pl.BlockSpec((NW, RB), lambda i: (0, i)),
            pl.BlockSpec((NW, RB), lambda i: (0, i + NPAD // RB)),
            pl.BlockSpec((1, 16), lambda i: (0, 0)),
            pl.BlockSpec((16, 40), lambda i: (0, 0)),
        ],
        out_specs=pl.BlockSpec((RB, 40), lambda i: (i, 0)),
        out_shape=jax.ShapeDtypeStruct((NPAD, 40), jnp.float32),
    )(p0, p1, degp, degp, b1, w2)


def _t3_body(q0_ref, q1_ref, dpd_ref, b2_ref, o_ref):
    nd = _norm_from(dpd_ref[...])
    o_ref[...] = (q0_ref[...] + q1_ref[...]) * nd[:, None] + b2_ref[...]


def _t3(q0, q1, degp, b2):
    return pl.pallas_call(
        _t3_body,
        grid=(NPAD // RB,),
        in_specs=[
            pl.BlockSpec((RB, 40), lambda i: (i, 0)),
            pl.BlockSpec((RB, 40), lambda i: (i, 0)),
            pl.BlockSpec((NW, RB), lambda i: (0, i + NPAD // RB)),
            pl.BlockSpec((1, 40), lambda i: (0, 0)),
        ],
        out_specs=pl.BlockSpec((RB, 40), lambda i: (i, 0)),
        out_shape=jax.ShapeDtypeStruct((N_NODES, 40), jnp.float32),
    )(q0, q1, degp, b2)


def kernel(features, edge_index, W1, b1, W2, b2):
    e2d = jnp.reshape(edge_index.astype(jnp.int32), (2, EROWS, 125))
    pad3 = (N_NODES + (jnp.arange(EROWS * 3, dtype=jnp.int32) % (NPAD - N_NODES))
            ).reshape(EROWS, 3)
    e3 = jnp.concatenate(
        [e2d, jnp.broadcast_to(pad3[None], (2, EROWS, 3))], axis=2)

    degp = _deg(e3)
    hs1 = jnp.reshape(_t1(features, W1, degp), (NPAD, 16))
    p0, p1 = _agg16(hs1, e3)
    hs2 = _t2(jnp.reshape(p0, (NPAD * 16 // 128, 128)),
              jnp.reshape(p1, (NPAD * 16 // 128, 128)),
              degp, jnp.reshape(b1, (1, 16)), W2)
    q0, q1 = _agg40(hs2, e3)
    return _t3(q0, q1, degp, jnp.reshape(b2, (1, 40)))


# speedup vs baseline: 16.7276x; 1.0728x over previous
import jax
import jax.numpy as jnp
from jax import lax
from jax.experimental import pallas as pl
from jax.experimental.pallas import tpu as pltpu
from jax.experimental.pallas import tpu_sc as plsc

N_NODES = 10000
N_EDGES = 160000
NPAD = 10240
DOFF = 10240
HIST = 2 * NPAD
NC = 2
NS = 16
NW = NC * NS
EB = 128
EROWS = 1280
BLK_PER_W = EROWS // NW
ROWS_PER_TILE = NPAD // NS
NBUF = 8

_mesh = lambda: plsc.VectorSubcoreMesh(
    core_axis_name="c", subcore_axis_name="s", num_cores=NC, num_subcores=NS)


def _deg_body(e_hbm, degp_hbm, sidx_v, didx_v, hist_v, sems):
    c = lax.axis_index("c")
    s = lax.axis_index("s")
    gid = c * NS + s
    row0 = gid * BLK_PER_W

    zero16f = jnp.zeros((16,), jnp.float32)
    ones16 = jnp.ones((16,), jnp.float32)

    pltpu.async_copy(e_hbm.at[0, pl.ds(row0, BLK_PER_W)], sidx_v, sems.at[0])
    pltpu.async_copy(e_hbm.at[1, pl.ds(row0, BLK_PER_W)], didx_v, sems.at[1])

    @pl.loop(0, HIST // 16)
    def _zero(i):
        hist_v[pl.ds(i * 16, 16)] = zero16f

    pltpu.make_async_copy(
        e_hbm.at[0, pl.ds(row0, BLK_PER_W)], sidx_v, sems.at[0]).wait()
    pltpu.make_async_copy(
        e_hbm.at[1, pl.ds(row0, BLK_PER_W)], didx_v, sems.at[1]).wait()

    @pl.loop(0, BLK_PER_W, unroll=2)
    def _scat(r):
        for k in range(EB // 16):
            sidx = sidx_v[r, pl.ds(k * 16, 16)]
            plsc.addupdate_scatter(hist_v, [sidx], ones16)
            didx = didx_v[r, pl.ds(k * 16, 16)] + DOFF
            plsc.addupdate_scatter(hist_v, [didx], ones16)

    pltpu.sync_copy(hist_v, degp_hbm.at[gid])


def _deg(e3):
    return pl.kernel(
        _deg_body,
        out_type=jax.ShapeDtypeStruct((NW, HIST), jnp.float32),
        mesh=_mesh(),
        compiler_params=pltpu.CompilerParams(needs_layout_passes=False),
        scratch_types=[
            pltpu.VMEM((BLK_PER_W, EB), jnp.int32),
            pltpu.VMEM((BLK_PER_W, EB), jnp.int32),
            pltpu.VMEM((HIST,), jnp.float32),
            pltpu.SemaphoreType.DMA((2,)),
        ],
    )(e3)


def _make_agg(D):
    nslices = (D + 15) // 16
    offs = [min(k * 16, D - 16) for k in range(nslices)]

    def body(hs_hbm, e_hbm, out0, out1,
             sidx_v, didx_v, rows_v, zbuf_v, agg_sh, isem, gsem, ssem):
        c = lax.axis_index("c")
        s = lax.axis_index("s")
        gid = c * NS + s
        row0 = gid * BLK_PER_W
        zero16f = jnp.zeros((16,), jnp.float32)

        pltpu.async_copy(e_hbm.at[0, pl.ds(row0, BLK_PER_W)], sidx_v, isem.at[0])
        pltpu.async_copy(e_hbm.at[1, pl.ds(row0, BLK_PER_W)], didx_v, isem.at[1])

        @pl.loop(0, ROWS_PER_TILE)
        def _z(i):
            for off in offs:
                zbuf_v[i, pl.ds(off, 16)] = zero16f

        pltpu.sync_copy(zbuf_v, agg_sh.at[pl.ds(s * ROWS_PER_TILE, ROWS_PER_TILE)])
        pltpu.make_async_copy(
            e_hbm.at[0, pl.ds(row0, BLK_PER_W)], sidx_v, isem.at[0]).wait()
        pltpu.make_async_copy(
            e_hbm.at[1, pl.ds(row0, BLK_PER_W)], didx_v, isem.at[1]).wait()
        plsc.subcore_barrier()

        def gather_start(j, b):
            pltpu.async_copy(hs_hbm.at[sidx_v.at[j]], rows_v.at[b], gsem.at[b])

        def gather_wait(j, b):
            pltpu.make_async_copy(
                hs_hbm.at[sidx_v.at[j]], rows_v.at[b], gsem.at[b]).wait()

        def scatter_start(j, b):
            pltpu.async_copy(
                rows_v.at[b], agg_sh.at[didx_v.at[j]], ssem.at[b], add=True)

        def scatter_wait(j, b):
            pltpu.make_async_copy(
                rows_v.at[b], agg_sh.at[didx_v.at[j]], ssem.at[b]).wait()

        for b in range(NBUF - 1):
            gather_start(b, b)

        @pl.loop(0, BLK_PER_W, step=NBUF)
        def _blk(j):
            for b in range(NBUF):
                jj = j + b
                gather_wait(jj, b)
                scatter_start(jj, b)
                nb = (b + NBUF - 1) % NBUF

                @pl.when(jj >= 1)
                def _():
                    scatter_wait(jj - 1, nb)

                @pl.when(jj + NBUF - 1 < BLK_PER_W)
                def _():
                    gather_start(jj + NBUF - 1, nb)

        scatter_wait(BLK_PER_W - 1, (BLK_PER_W - 1) % NBUF)
        plsc.subcore_barrier()

        stripe = pl.ds(s * ROWS_PER_TILE, ROWS_PER_TILE)
        pltpu.sync_copy(agg_sh.at[stripe], zbuf_v)

        @pl.when(c == 0)
        def _():
            pltpu.sync_copy(zbuf_v, out0.at[stripe])

        @pl.when(c == 1)
        def _():
            pltpu.sync_copy(zbuf_v, out1.at[stripe])

    def run(hs, e3):
        return pl.kernel(
            body,
            out_type=(jax.ShapeDtypeStruct((NPAD, D), jnp.float32),
                      jax.ShapeDtypeStruct((NPAD, D), jnp.float32)),
            mesh=_mesh(),
            compiler_params=pltpu.CompilerParams(use_tc_tiling_on_sc=False),
            scratch_types=[
                pltpu.VMEM((BLK_PER_W, EB), jnp.int32),
                pltpu.VMEM((BLK_PER_W, EB), jnp.int32),
                pltpu.VMEM((NBUF, EB, D), jnp.float32),
                pltpu.VMEM((ROWS_PER_TILE, D), jnp.float32),
                pltpu.VMEM_SHARED((NPAD, D), jnp.float32),
                pltpu.SemaphoreType.DMA((2,)),
                pltpu.SemaphoreType.DMA((NBUF,)),
                pltpu.SemaphoreType.DMA((NBUF,)),
            ],
        )(hs, e3)

    return run


_agg16 = _make_agg(16)
_agg40 = _make_agg(40)


RB = 2560


def _norm_from(degp_blk):
    deg = jnp.sum(degp_blk, axis=0)
    return jnp.where(deg > 0, lax.rsqrt(jnp.maximum(deg, 1.0)), 0.0)


def _rowmask(i, rows):
    row = i * RB + lax.broadcasted_iota(jnp.int32, (rows, 1), 0)
    return row < N_NODES


R16 = RB * 16 // 128
R40 = RB * 40 // 128


def _t1_body(x_ref, w_ref, degp_ref, o_ref):
    i = pl.program_id(0)
    ns = _norm_from(degp_ref[...])
    h = jnp.dot(x_ref[...], w_ref[...], preferred_element_type=jnp.float32)
    hs = jnp.where(_rowmask(i, RB), h * ns[:, None], 0.0)
    hs3 = jnp.reshape(hs, (R16, 8, 16))
    for g in range(8):
        o_ref[:, g * 16:(g + 1) * 16] = hs3[:, g, :]


def _t1(x, w1, degp):
    return pl.pallas_call(
        _t1_body,
        grid=(NPAD // RB,),
        in_specs=[
            pl.BlockSpec((RB, 256), lambda i: (i, 0)),
            pl.BlockSpec((256, 16), lambda i: (0, 0)),
            pl.BlockSpec((NW, RB), lambda i: (0, i)),
        ],
        out_specs=pl.BlockSpec((R16, 128), lambda i: (i, 0)),
        out_shape=jax.ShapeDtypeStruct((NPAD * 16 // 128, 128), jnp.float32),
    )(x, w1, degp)


def _t2_body(p0_ref, p1_ref, dps_ref, dpd_ref, b1_ref, w2_ref, o_ref, scr):
    i = pl.program_id(0)
    ns = _norm_from(dps_ref[...])
    nd = _norm_from(dpd_ref[...])
    psum = p0_ref[...] + p1_ref[...]
    for g in range(8):
        scr[:, g, :] = psum[:, g * 16:(g + 1) * 16]
    p = jnp.reshape(scr[...], (RB, 16))
    agg = p * nd[:, None] + b1_ref[...]
    y = jnp.maximum(agg, 0.0)
    h2 = jnp.dot(y, w2_ref[...], preferred_element_type=jnp.float32)
    o_ref[...] = jnp.where(_rowmask(i, RB), h2 * ns[:, None], 0.0)


def _t2(p0, p1, degp, b1, w2):
    return pl.pallas_call(
        _t2_body,
        grid=(NPAD // RB,),
        in_specs=[
            pl.BlockSpec((R16, 128), lambda i: (i, 0)),
            pl.BlockSpec((R16, 128), lambda i: (i, 0)),
            pl.BlockSpec((NW, RB), lambda i: (0, i)),
            pl.BlockSpec((NW, RB), lambda i: (0, i + NPAD // RB)),
            pl.BlockSpec((1, 16), lambda i: (0, 0)),
            pl.BlockSpec((16, 40), lambda i: (0, 0)),
        ],
        out_specs=pl.BlockSpec((RB, 40), lambda i: (i, 0)),
        out_shape=jax.ShapeDtypeStruct((NPAD, 40), jnp.float32),
        scratch_shapes=[pltpu.VMEM((R16, 8, 16), jnp.float32)],
    )(p0, p1, degp, degp, b1, w2)


def _t3_body(q0_ref, q1_ref, dpd_ref, b2_ref, o_ref):
    nd = _norm_from(dpd_ref[...])
    o_ref[...] = (q0_ref[...] + q1_ref[...]) * nd[:, None] + b2_ref[...]


def _t3(q0, q1, degp, b2):
    return pl.pallas_call(
        _t3_body,
        grid=(NPAD // RB,),
        in_specs=[
            pl.BlockSpec((RB, 40), lambda i: (i, 0)),
            pl.BlockSpec((RB, 40), lambda i: (i, 0)),
            pl.BlockSpec((NW, RB), lambda i: (0, i + NPAD // RB)),
            pl.BlockSpec((1, 40), lambda i: (0, 0)),
        ],
        out_specs=pl.BlockSpec((RB, 40), lambda i: (i, 0)),
        out_shape=jax.ShapeDtypeStruct((N_NODES, 40), jnp.float32),
    )(q0, q1, degp, b2)


def kernel(features, edge_index, W1, b1, W2, b2):
    e2d = jnp.reshape(edge_index.astype(jnp.int32), (2, EROWS, 125))
    pad3 = (N_NODES + (jnp.arange(EROWS * 3, dtype=jnp.int32) % (NPAD - N_NODES))
            ).reshape(EROWS, 3)
    e3 = jnp.concatenate(
        [e2d, jnp.broadcast_to(pad3[None], (2, EROWS, 3))], axis=2)

    degp = _deg(e3)
    hs1 = jnp.reshape(_t1(features, W1, degp), (NPAD, 16))
    p0, p1 = _agg16(hs1, e3)
    hs2 = _t2(jnp.reshape(p0, (NPAD * 16 // 128, 128)),
              jnp.reshape(p1, (NPAD * 16 // 128, 128)),
              degp, jnp.reshape(b1, (1, 16)), W2)
    q0, q1 = _agg40(hs2, e3)
    return _t3(q0, q1, degp, jnp.reshape(b2, (1, 40)))
